# Initial kernel scaffold; baseline (speedup 1.0000x reference)
import jax
import jax.numpy as jnp
from jax.experimental import pallas as pl


def kernel(pred, target, train_mask):
    raise NotImplementedError("write your pallas kernel here")


# speedup vs baseline: 27.2769x; 27.2769x over previous
import jax
import jax.numpy as jnp
from jax.experimental import pallas as pl
from jax.experimental.pallas import tpu as pltpu

_NCHUNK = 4
_NBIN = 16
_NHIST = 3
_NPASS = _NHIST + 1
_LOSS_HI = 16.13

_LO, _W, _NPOS, _LPOS, _NNEG, _K, _KEFF, _KREM, _SUM, _CNT = range(10)


def _body(pred_ref, t_ref, m_ref, out_ref, loss_ref, sm, hist):
    p = pl.program_id(0)
    c = pl.program_id(1)

    @pl.when((p == 0) & (c == 0))
    def _init():
        sm[_LO] = 0.0
        sm[_W] = _LOSS_HI / _NBIN
        sm[_NPOS] = 0.0
        sm[_LPOS] = 0.0
        sm[_NNEG] = 0.0
        sm[_SUM] = 0.0
        sm[_CNT] = 0.0
        for b in range(_NBIN):
            hist[b] = 0.0

    @pl.when(p == 0)
    def _bce():
        pr = jnp.clip(pred_ref[...], 1e-7, 1.0 - 1e-7)
        t = t_ref[...]
        m = m_ref[...]
        tf = t.astype(jnp.float32)
        losses = -(tf * jnp.log(pr) + (1.0 - tf) * jnp.log(1.0 - pr))
        pos = (t * m) > 0
        neg = ((1 - t) * m) > 0
        sm[_NPOS] += jnp.sum(pos.astype(jnp.float32))
        sm[_NNEG] += jnp.sum(neg.astype(jnp.float32))
        sm[_LPOS] += jnp.sum(jnp.where(pos, losses, 0.0))
        loss_ref[pl.ds(c, 1)] = jnp.where(neg, losses, -1.0)

    v = loss_ref[pl.ds(c, 1)]

    @pl.when(p < _NHIST)
    def _histogram():
        lo = sm[_LO]
        inv_w = 1.0 / sm[_W]
        idx = jnp.where(v >= lo, ((v - lo) * inv_w).astype(jnp.int32), -1)
        for b in range(_NBIN):
            hist[b] += jnp.sum((idx == b).astype(jnp.float32))

    @pl.when((p < _NHIST) & (c == _NCHUNK - 1))
    def _select():
        @pl.when(p == 0)
        def _set_k():
            npos = sm[_NPOS]
            nneg = sm[_NNEG]
            k = jnp.where(npos > 0.0, jnp.minimum(nneg, 3.0 * npos), 100.0)
            sm[_K] = k
            sm[_KEFF] = jnp.minimum(k, nneg)
            sm[_KREM] = jnp.minimum(k, nneg)

        k_rem = sm[_KREM]
        running = jnp.float32(0.0)
        found = jnp.bool_(False)
        b_sel = jnp.float32(0.0)
        above = jnp.float32(0.0)
        for b in range(_NBIN - 1, -1, -1):
            hb = hist[b]
            running_new = running + hb
            crossed = jnp.logical_and(
                jnp.logical_not(found),
                jnp.logical_and(running_new >= k_rem, k_rem > 0.0))
            b_sel = jnp.where(crossed, jnp.float32(b), b_sel)
            above = jnp.where(crossed, running, above)
            found = jnp.logical_or(found, crossed)
            running = running_new
        w = sm[_W]
        sm[_LO] = sm[_LO] + b_sel * w
        sm[_W] = w / _NBIN
        sm[_KREM] = k_rem - above
        for b in range(_NBIN):
            hist[b] = 0.0

    @pl.when(p == _NHIST)
    def _final_sum():
        theta = sm[_LO]
        sel = v >= theta
        sm[_SUM] += jnp.sum(jnp.where(sel, v, 0.0))
        sm[_CNT] += jnp.sum(sel.astype(jnp.float32))

    @pl.when((p == _NHIST) & (c == _NCHUNK - 1))
    def _finish():
        theta = sm[_LO]
        k = sm[_K]
        k_eff = sm[_KEFF]
        nneg = sm[_NNEG]
        loss_neg = sm[_SUM] - (sm[_CNT] - k_eff) * theta
        loss_neg = jnp.where(k_eff > 0.0, loss_neg, 0.0)
        loss_neg = loss_neg + jnp.where(k > nneg, (k - nneg) * -1e30, 0.0)
        out_ref[0, 0] = (sm[_LPOS] + loss_neg) / (sm[_NPOS] + k)


def _run(pred, target, train_mask, interpret=False):
    n, h, w = pred.shape
    in_map = lambda p, c: (jnp.where(p == 0, c, 0), 0, 0)
    out = pl.pallas_call(
        _body,
        grid=(_NPASS, _NCHUNK),
        in_specs=[
            pl.BlockSpec((1, h, w), in_map),
            pl.BlockSpec((1, h, w), in_map),
            pl.BlockSpec((1, h, w), in_map),
        ],
        out_specs=pl.BlockSpec((1, 1), lambda p, c: (0, 0),
                               memory_space=pltpu.SMEM),
        out_shape=jax.ShapeDtypeStruct((1, 1), jnp.float32),
        scratch_shapes=[
            pltpu.VMEM((n, h, w), jnp.float32),
            pltpu.SMEM((16,), jnp.float32),
            pltpu.SMEM((_NBIN,), jnp.float32),
        ],
        compiler_params=pltpu.CompilerParams(
            dimension_semantics=("arbitrary", "arbitrary")),
        interpret=interpret,
    )(pred, target, train_mask)
    return out[0, 0]


def kernel(pred, target, train_mask):
    return _run(pred, target, train_mask)
